# baseline (speedup 1.0000x reference)
import jax
import jax.numpy as jnp
from jax.experimental import pallas as pl

N_PTS = 128
MIN_DEPTH = 0.1
MAX_DEPTH = 100.0
BM = 2048


def _raysample_kernel(od_ref, r_ref, z_ref, pts_ref, lens_ref):
    pts_ref[...] = jnp.dot(od_ref[...], r_ref[...],
                           preferred_element_type=jnp.float32)
    lens_ref[...] = jnp.broadcast_to(z_ref[...], lens_ref.shape)


@jax.jit
def kernel(origins, directions):
    B = origins.shape[0]
    z = jnp.linspace(MIN_DEPTH, MAX_DEPTH, N_PTS, dtype=jnp.float32)
    eye = jnp.eye(3, dtype=jnp.float32)
    p = jnp.tile(eye, (1, N_PTS))
    q = (z[None, :, None] * eye[:, None, :]).reshape(3, N_PTS * 3)
    r = jnp.concatenate([p, q, jnp.zeros((2, N_PTS * 3), jnp.float32)], axis=0)
    od = jnp.concatenate(
        [origins, directions, jnp.zeros((B, 2), jnp.float32)], axis=1)

    pts, lens = pl.pallas_call(
        _raysample_kernel,
        grid=(B // BM,),
        in_specs=[
            pl.BlockSpec((BM, 8), lambda i: (i, 0)),
            pl.BlockSpec((8, N_PTS * 3), lambda i: (0, 0)),
            pl.BlockSpec((1, N_PTS), lambda i: (0, 0)),
        ],
        out_specs=[
            pl.BlockSpec((BM, N_PTS * 3), lambda i: (i, 0)),
            pl.BlockSpec((BM, N_PTS), lambda i: (i, 0)),
        ],
        out_shape=[
            jax.ShapeDtypeStruct((B, N_PTS * 3), jnp.float32),
            jax.ShapeDtypeStruct((B, N_PTS), jnp.float32),
        ],
    )(od, r, z.reshape(1, N_PTS))

    return pts.reshape(B, N_PTS, 3), lens.reshape(B, N_PTS, 1)


# speedup vs baseline: 6.3839x; 6.3839x over previous
import jax
import jax.numpy as jnp
from jax.experimental import pallas as pl

N_PTS = 128
MIN_DEPTH = 0.1
MAX_DEPTH = 100.0
BM = 2048


def _raysample_kernel(o_ref, d_ref, z_ref, pts_ref, lens_ref):
    z = z_ref[...]
    for c in range(3):
        pts_ref[c] = o_ref[:, c:c + 1] + d_ref[:, c:c + 1] * z
    lens_ref[...] = jnp.broadcast_to(z, lens_ref.shape)


@jax.jit
def kernel(origins, directions):
    B = origins.shape[0]
    z = jnp.linspace(MIN_DEPTH, MAX_DEPTH, N_PTS, dtype=jnp.float32)

    planes, lens = pl.pallas_call(
        _raysample_kernel,
        grid=(B // BM,),
        in_specs=[
            pl.BlockSpec((BM, 3), lambda i: (i, 0)),
            pl.BlockSpec((BM, 3), lambda i: (i, 0)),
            pl.BlockSpec((1, N_PTS), lambda i: (0, 0)),
        ],
        out_specs=[
            pl.BlockSpec((3, BM, N_PTS), lambda i: (0, i, 0)),
            pl.BlockSpec((BM, N_PTS), lambda i: (i, 0)),
        ],
        out_shape=[
            jax.ShapeDtypeStruct((3, B, N_PTS), jnp.float32),
            jax.ShapeDtypeStruct((B, N_PTS), jnp.float32),
        ],
    )(origins, directions, z.reshape(1, N_PTS))

    return jnp.transpose(planes, (1, 2, 0)), lens.reshape(B, N_PTS, 1)
